# Initial kernel scaffold; baseline (speedup 1.0000x reference)
import jax
import jax.numpy as jnp
from jax.experimental import pallas as pl


def kernel(xyz, sa1_W0, sa1_b0, sa1_g0, sa1_be0, sa1_W1, sa1_b1, sa1_g1, sa1_be1, sa1_W2, sa1_b2, sa1_g2, sa1_be2, sa2_W0, sa2_b0, sa2_g0, sa2_be0, sa2_W1, sa2_b1, sa2_g1, sa2_be1, sa2_W2, sa2_b2, sa2_g2, sa2_be2):
    raise NotImplementedError("write your pallas kernel here")


# speedup vs baseline: 9.9859x; 9.9859x over previous
import jax
import jax.numpy as jnp
from jax.experimental import pallas as pl

_F32 = jnp.float32
_RADIUS2 = 0.4 ** 2


def _stats_block(y):
    red = tuple(range(y.ndim - 1))
    s = jnp.sum(y, axis=red)
    sq = jnp.sum(y * y, axis=red)
    rid = jax.lax.broadcasted_iota(jnp.int32, (8, s.shape[0]), 0)
    return jnp.where(rid == 0, s[None, :], 0.0) + jnp.where(rid == 1, sq[None, :], 0.0)


def _accum_stats(st_ref, block):
    @pl.when(pl.program_id(0) == 0)
    def _():
        st_ref[...] = block

    @pl.when(pl.program_id(0) != 0)
    def _():
        st_ref[...] = st_ref[...] + block


def _scale_shift(st_ref, g_ref, be_ref, n_rows):
    inv_n = 1.0 / n_rows
    mean = st_ref[0:1, :] * inv_n
    var = st_ref[1:2, :] * inv_n - mean * mean
    sc = g_ref[...] * jax.lax.rsqrt(var + 1e-5)
    sh = be_ref[...] - mean * sc
    return sc, sh


def _group_body(px_ref, py_ref, pz_ref, w_ref, b_ref,
                y_ref, ncx_ref, ncy_ref, ncz_ref, st_ref):
    x = px_ref[...]
    y = py_ref[...]
    z = pz_ref[...]
    cb = x.shape[0]
    lane = jax.lax.broadcasted_iota(jnp.int32, (cb, 32), 1)

    dist = jnp.full((cb, 32), 1e10, _F32)
    far = jnp.zeros((cb, 1), jnp.int32)
    cents = []
    for i in range(8):
        oh = lane == far
        cx = jnp.sum(jnp.where(oh, x, 0.0), axis=1, keepdims=True)
        cy = jnp.sum(jnp.where(oh, y, 0.0), axis=1, keepdims=True)
        cz = jnp.sum(jnp.where(oh, z, 0.0), axis=1, keepdims=True)
        cents.append((cx, cy, cz))
        d = (x - cx) ** 2 + (y - cy) ** 2 + (z - cz) ** 2
        dist = jnp.minimum(dist, d)
        if i < 7:
            mx = jnp.max(dist, axis=1, keepdims=True)
            far = jnp.min(jnp.where(dist == mx, lane, 32), axis=1, keepdims=True)

    tri = (jax.lax.broadcasted_iota(jnp.int32, (32, 32), 0)
           <= jax.lax.broadcasted_iota(jnp.int32, (32, 32), 1)).astype(_F32)
    slots = (jax.lax.broadcasted_iota(jnp.int32, (1, 8, 1), 1) + 1).astype(_F32)
    wx = w_ref[0:1, :].astype(jnp.bfloat16).astype(_F32)[None]
    wy = w_ref[1:2, :].astype(jnp.bfloat16).astype(_F32)[None]
    wz = w_ref[2:3, :].astype(jnp.bfloat16).astype(_F32)[None]
    bb = b_ref[...][None]
    yparts = []
    for j in range(8):
        cx, cy, cz = cents[j]
        d = (x - cx) ** 2 + (y - cy) ** 2 + (z - cz) ** 2
        valid = d <= _RADIUS2
        rank = jnp.dot(valid.astype(_F32), tri, preferred_element_type=_F32)
        cond = valid[:, None, :] & (rank[:, None, :] == slots)
        cand = jnp.where(cond, lane[:, None, :], 32)
        idx = jnp.min(cand, axis=2)
        first = idx[:, 0:1]
        idx = jnp.where(idx == 32, first, idx)
        oh8 = idx[:, :, None] == lane[:, None, :]
        gx = jnp.sum(jnp.where(oh8, x[:, None, :], 0.0), axis=2) - cx
        gy = jnp.sum(jnp.where(oh8, y[:, None, :], 0.0), axis=2) - cy
        gz = jnp.sum(jnp.where(oh8, z[:, None, :], 0.0), axis=2) - cz
        gxb = gx.astype(jnp.bfloat16).astype(_F32)
        gyb = gy.astype(jnp.bfloat16).astype(_F32)
        gzb = gz.astype(jnp.bfloat16).astype(_F32)
        yparts.append(gxb[:, :, None] * wx + gyb[:, :, None] * wy
                      + gzb[:, :, None] * wz + bb)
    y1 = jnp.concatenate(yparts, axis=1)
    y_ref[...] = y1
    ncx_ref[...] = jnp.concatenate([c[0] for c in cents], axis=1)
    ncy_ref[...] = jnp.concatenate([c[1] for c in cents], axis=1)
    ncz_ref[...] = jnp.concatenate([c[2] for c in cents], axis=1)
    _accum_stats(st_ref, _stats_block(y1))


def _make_mlp(n_rows):
    def body(y_ref, st_ref, g_ref, be_ref, wt_ref, b_ref, yo_ref, sto_ref):
        sc, sh = _scale_shift(st_ref, g_ref, be_ref, n_rows)
        zz = jnp.maximum(y_ref[...] * sc + sh, 0.0)
        yo = jax.lax.dot_general(zz.astype(jnp.bfloat16), wt_ref[...].astype(jnp.bfloat16),
                                 (((1,), (0,)), ((), ())),
                                 preferred_element_type=_F32) + b_ref[...]
        yo_ref[...] = yo
        _accum_stats(sto_ref, _stats_block(yo))
    return body


def _make_trans(n_rows):
    def body(y_ref, st_ref, g_ref, be_ref, ncx_ref, ncy_ref, ncz_ref,
             wx_ref, wy_ref, wz_ref, wt_ref, b_ref, yo_ref, sto_ref):
        sc, sh = _scale_shift(st_ref, g_ref, be_ref, n_rows)
        zz = jnp.maximum(y_ref[...] * sc + sh, 0.0)
        cb = zz.shape[0]
        pooled = jnp.max(zz.reshape(cb, 8, 8, 128), axis=2)
        p2 = pooled.reshape(cb * 8, 128)
        yo = jax.lax.dot_general(p2.astype(jnp.bfloat16), wt_ref[...].astype(jnp.bfloat16),
                                 (((1,), (0,)), ((), ())),
                                 preferred_element_type=_F32)
        ncxb = ncx_ref[...].astype(jnp.bfloat16).astype(_F32)
        ncyb = ncy_ref[...].astype(jnp.bfloat16).astype(_F32)
        nczb = ncz_ref[...].astype(jnp.bfloat16).astype(_F32)
        wxb = wx_ref[...].astype(jnp.bfloat16).astype(_F32)
        wyb = wy_ref[...].astype(jnp.bfloat16).astype(_F32)
        wzb = wz_ref[...].astype(jnp.bfloat16).astype(_F32)
        yo = (yo + ncxb * wxb + ncyb * wyb + nczb * wzb + b_ref[...])
        yo_ref[...] = yo
        _accum_stats(sto_ref, _stats_block(yo))
    return body


def _make_final(n_rows):
    def body(y_ref, st_ref, g_ref, be_ref, o_ref):
        sc, sh = _scale_shift(st_ref, g_ref, be_ref, n_rows)
        zz = jnp.maximum(y_ref[...] * sc + sh, 0.0)
        r = zz.shape[0] // 8
        o_ref[...] = jnp.max(zz.reshape(r, 8, zz.shape[1]), axis=1)
    return body


def kernel(xyz, sa1_W0, sa1_b0, sa1_g0, sa1_be0, sa1_W1, sa1_b1, sa1_g1, sa1_be1,
           sa1_W2, sa1_b2, sa1_g2, sa1_be2, sa2_W0, sa2_b0, sa2_g0, sa2_be0,
           sa2_W1, sa2_b1, sa2_g1, sa2_be1, sa2_W2, sa2_b2, sa2_g2, sa2_be2):
    bs, nv, k, _ = xyz.shape
    B = bs * nv
    xt = jnp.transpose(xyz.reshape(B, k, 3), (2, 0, 1))
    px, py, pz = xt[0], xt[1], xt[2]

    cb_a = 128
    y1, ncx, ncy, ncz, st1 = pl.pallas_call(
        _group_body,
        grid=(B // cb_a,),
        in_specs=[
            pl.BlockSpec((cb_a, 32), lambda i: (i, 0)),
            pl.BlockSpec((cb_a, 32), lambda i: (i, 0)),
            pl.BlockSpec((cb_a, 32), lambda i: (i, 0)),
            pl.BlockSpec((3, 64), lambda i: (0, 0)),
            pl.BlockSpec((1, 64), lambda i: (0, 0)),
        ],
        out_specs=[
            pl.BlockSpec((cb_a, 64, 64), lambda i: (i, 0, 0)),
            pl.BlockSpec((cb_a, 8), lambda i: (i, 0)),
            pl.BlockSpec((cb_a, 8), lambda i: (i, 0)),
            pl.BlockSpec((cb_a, 8), lambda i: (i, 0)),
            pl.BlockSpec((8, 64), lambda i: (0, 0)),
        ],
        out_shape=[
            jax.ShapeDtypeStruct((B, 64, 64), _F32),
            jax.ShapeDtypeStruct((B, 8), _F32),
            jax.ShapeDtypeStruct((B, 8), _F32),
            jax.ShapeDtypeStruct((B, 8), _F32),
            jax.ShapeDtypeStruct((8, 64), _F32),
        ],
    )(px, py, pz, jnp.transpose(sa1_W0), sa1_b0.reshape(1, 64))

    def mlp(yprev, st, g, be, W, b, n_rows, rb):
        R, cin = yprev.shape
        cout = W.shape[0]
        return pl.pallas_call(
            _make_mlp(n_rows),
            grid=(R // rb,),
            in_specs=[
                pl.BlockSpec((rb, cin), lambda i: (i, 0)),
                pl.BlockSpec((8, cin), lambda i: (0, 0)),
                pl.BlockSpec((1, cin), lambda i: (0, 0)),
                pl.BlockSpec((1, cin), lambda i: (0, 0)),
                pl.BlockSpec((cin, cout), lambda i: (0, 0)),
                pl.BlockSpec((1, cout), lambda i: (0, 0)),
            ],
            out_specs=[
                pl.BlockSpec((rb, cout), lambda i: (i, 0)),
                pl.BlockSpec((8, cout), lambda i: (0, 0)),
            ],
            out_shape=[
                jax.ShapeDtypeStruct((R, cout), _F32),
                jax.ShapeDtypeStruct((8, cout), _F32),
            ],
        )(yprev, st, g.reshape(1, cin), be.reshape(1, cin),
          jnp.transpose(W), b.reshape(1, cout))

    n1 = float(B * 64)
    n2 = float(B * 8)

    y2, st2 = mlp(y1.reshape(B * 64, 64), st1, sa1_g0, sa1_be0, sa1_W1, sa1_b1, n1, 8192)
    y3, st3 = mlp(y2, st2, sa1_g1, sa1_be1, sa1_W2, sa1_b2, n1, 8192)

    cb_c = 128
    y4, st4 = pl.pallas_call(
        _make_trans(n1),
        grid=(B // cb_c,),
        in_specs=[
            pl.BlockSpec((cb_c, 64, 128), lambda i: (i, 0, 0)),
            pl.BlockSpec((8, 128), lambda i: (0, 0)),
            pl.BlockSpec((1, 128), lambda i: (0, 0)),
            pl.BlockSpec((1, 128), lambda i: (0, 0)),
            pl.BlockSpec((cb_c * 8, 1), lambda i: (i, 0)),
            pl.BlockSpec((cb_c * 8, 1), lambda i: (i, 0)),
            pl.BlockSpec((cb_c * 8, 1), lambda i: (i, 0)),
            pl.BlockSpec((1, 128), lambda i: (0, 0)),
            pl.BlockSpec((1, 128), lambda i: (0, 0)),
            pl.BlockSpec((1, 128), lambda i: (0, 0)),
            pl.BlockSpec((128, 128), lambda i: (0, 0)),
            pl.BlockSpec((1, 128), lambda i: (0, 0)),
        ],
        out_specs=[
            pl.BlockSpec((cb_c * 8, 128), lambda i: (i, 0)),
            pl.BlockSpec((8, 128), lambda i: (0, 0)),
        ],
        out_shape=[
            jax.ShapeDtypeStruct((B * 8, 128), _F32),
            jax.ShapeDtypeStruct((8, 128), _F32),
        ],
    )(y3.reshape(B, 64, 128), st3, sa1_g2.reshape(1, 128), sa1_be2.reshape(1, 128),
      ncx.reshape(B * 8, 1), ncy.reshape(B * 8, 1), ncz.reshape(B * 8, 1),
      sa2_W0[:, 0].reshape(1, 128), sa2_W0[:, 1].reshape(1, 128),
      sa2_W0[:, 2].reshape(1, 128), jnp.transpose(sa2_W0[:, 3:]),
      sa2_b0.reshape(1, 128))

    y5, st5 = mlp(y4, st4, sa2_g0, sa2_be0, sa2_W1, sa2_b1, n2, 4096)
    y6, st6 = mlp(y5, st5, sa2_g1, sa2_be1, sa2_W2, sa2_b2, n2, 2048)

    cb_d = 256
    out = pl.pallas_call(
        _make_final(n2),
        grid=(B // cb_d,),
        in_specs=[
            pl.BlockSpec((cb_d * 8, 512), lambda i: (i, 0)),
            pl.BlockSpec((8, 512), lambda i: (0, 0)),
            pl.BlockSpec((1, 512), lambda i: (0, 0)),
            pl.BlockSpec((1, 512), lambda i: (0, 0)),
        ],
        out_specs=pl.BlockSpec((cb_d, 512), lambda i: (i, 0)),
        out_shape=jax.ShapeDtypeStruct((B, 512), _F32),
    )(y6, st6, sa2_g2.reshape(1, 512), sa2_be2.reshape(1, 512))

    return out.reshape(bs, nv, 512)


# speedup vs baseline: 12.0845x; 1.2102x over previous
import jax
import jax.numpy as jnp
from jax.experimental import pallas as pl

_F32 = jnp.float32
_RADIUS2 = 0.4 ** 2


def _stats_block(y):
    red = tuple(range(y.ndim - 1))
    s = jnp.sum(y, axis=red)
    sq = jnp.sum(y * y, axis=red)
    rid = jax.lax.broadcasted_iota(jnp.int32, (8, s.shape[0]), 0)
    return jnp.where(rid == 0, s[None, :], 0.0) + jnp.where(rid == 1, sq[None, :], 0.0)


def _accum_stats(st_ref, block):
    @pl.when(pl.program_id(0) == 0)
    def _():
        st_ref[...] = block

    @pl.when(pl.program_id(0) != 0)
    def _():
        st_ref[...] = st_ref[...] + block


def _scale_shift(st_ref, g_ref, be_ref, n_rows):
    inv_n = 1.0 / n_rows
    mean = st_ref[0:1, :] * inv_n
    var = st_ref[1:2, :] * inv_n - mean * mean
    sc = g_ref[...] * jax.lax.rsqrt(var + 1e-5)
    sh = be_ref[...] - mean * sc
    return sc, sh


def _group_body(px_ref, py_ref, pz_ref, w_ref, b_ref,
                y_ref, ncx_ref, ncy_ref, ncz_ref, st_ref):
    x = px_ref[...]
    y = py_ref[...]
    z = pz_ref[...]
    cb = x.shape[0]
    lane = jax.lax.broadcasted_iota(jnp.int32, (cb, 32), 1)

    dist = jnp.full((cb, 32), 1e10, _F32)
    far = jnp.zeros((cb, 1), jnp.int32)
    cents = []
    for i in range(8):
        oh = lane == far
        cx = jnp.sum(jnp.where(oh, x, 0.0), axis=1, keepdims=True)
        cy = jnp.sum(jnp.where(oh, y, 0.0), axis=1, keepdims=True)
        cz = jnp.sum(jnp.where(oh, z, 0.0), axis=1, keepdims=True)
        cents.append((cx, cy, cz))
        d = (x - cx) ** 2 + (y - cy) ** 2 + (z - cz) ** 2
        dist = jnp.minimum(dist, d)
        if i < 7:
            mx = jnp.max(dist, axis=1, keepdims=True)
            far = jnp.min(jnp.where(dist == mx, lane, 32), axis=1, keepdims=True)

    x8 = jnp.concatenate([x] * 8, axis=1)
    y8 = jnp.concatenate([y] * 8, axis=1)
    z8 = jnp.concatenate([z] * 8, axis=1)
    cx8 = jnp.concatenate([jnp.broadcast_to(c[0], (cb, 32)) for c in cents], axis=1)
    cy8 = jnp.concatenate([jnp.broadcast_to(c[1], (cb, 32)) for c in cents], axis=1)
    cz8 = jnp.concatenate([jnp.broadcast_to(c[2], (cb, 32)) for c in cents], axis=1)
    d8 = (x8 - cx8) ** 2 + (y8 - cy8) ** 2 + (z8 - cz8) ** 2
    vf = (d8 <= _RADIUS2).astype(_F32)

    r0 = jax.lax.broadcasted_iota(jnp.int32, (256, 256), 0)
    r1 = jax.lax.broadcasted_iota(jnp.int32, (256, 256), 1)
    tri = ((r0 <= r1) & ((r0 // 32) == (r1 // 32))).astype(_F32)
    rank = jnp.dot(vf, tri, preferred_element_type=_F32)

    q0 = jax.lax.broadcasted_iota(jnp.int32, (1024, 32), 0)
    q1 = jax.lax.broadcasted_iota(jnp.int32, (1024, 32), 1)
    seg4 = (((q0 // 256) == (q1 // 8)) & (((q0 % 256) // 32) == (q1 % 8))).astype(_F32)

    ncx = jnp.concatenate([c[0] for c in cents], axis=1)
    ncy = jnp.concatenate([c[1] for c in cents], axis=1)
    ncz = jnp.concatenate([c[2] for c in cents], axis=1)

    wx = w_ref[0:1, :].astype(jnp.bfloat16).astype(_F32)[None]
    wy = w_ref[1:2, :].astype(jnp.bfloat16).astype(_F32)[None]
    wz = w_ref[2:3, :].astype(jnp.bfloat16).astype(_F32)[None]
    bb = b_ref[...][None]
    g0 = None
    ssum = None
    ssq = None
    for t in range(8):
        m = jnp.where(rank == float(t + 1), vf, 0.0)
        m3 = jnp.concatenate([m * x8, m * y8, m * z8, m], axis=1)
        gsel = jnp.dot(m3, seg4, preferred_element_type=_F32,
                       precision=jax.lax.Precision.HIGHEST)
        gx = gsel[:, 0:8]
        gy = gsel[:, 8:16]
        gz = gsel[:, 16:24]
        fnd = gsel[:, 24:32]
        if t == 0:
            g0 = (gx, gy, gz)
        else:
            gx = jnp.where(fnd > 0.5, gx, g0[0])
            gy = jnp.where(fnd > 0.5, gy, g0[1])
            gz = jnp.where(fnd > 0.5, gz, g0[2])
        gxb = (gx - ncx).astype(jnp.bfloat16).astype(_F32)
        gyb = (gy - ncy).astype(jnp.bfloat16).astype(_F32)
        gzb = (gz - ncz).astype(jnp.bfloat16).astype(_F32)
        yt = (gxb[:, :, None] * wx + gyb[:, :, None] * wy
              + gzb[:, :, None] * wz + bb)
        y_ref[:, t * 8:(t + 1) * 8, :] = yt
        ts = jnp.sum(yt, axis=(0, 1))
        tq = jnp.sum(yt * yt, axis=(0, 1))
        ssum = ts if ssum is None else ssum + ts
        ssq = tq if ssq is None else ssq + tq
    ncx_ref[...] = ncx
    ncy_ref[...] = ncy
    ncz_ref[...] = ncz
    rid = jax.lax.broadcasted_iota(jnp.int32, (8, 64), 0)
    blk = (jnp.where(rid == 0, ssum[None, :], 0.0)
           + jnp.where(rid == 1, ssq[None, :], 0.0))
    _accum_stats(st_ref, blk)


def _make_mlp(n_rows):
    def body(y_ref, st_ref, g_ref, be_ref, wt_ref, b_ref, yo_ref, sto_ref):
        sc, sh = _scale_shift(st_ref, g_ref, be_ref, n_rows)
        zz = jnp.maximum(y_ref[...] * sc + sh, 0.0)
        yo = jax.lax.dot_general(zz.astype(jnp.bfloat16), wt_ref[...].astype(jnp.bfloat16),
                                 (((1,), (0,)), ((), ())),
                                 preferred_element_type=_F32) + b_ref[...]
        yo_ref[...] = yo
        _accum_stats(sto_ref, _stats_block(yo))
    return body


def _make_trans(n_rows):
    def body(y_ref, st_ref, g_ref, be_ref, ncx_ref, ncy_ref, ncz_ref,
             wx_ref, wy_ref, wz_ref, wt_ref, b_ref, yo_ref, sto_ref):
        sc, sh = _scale_shift(st_ref, g_ref, be_ref, n_rows)
        zz = jnp.maximum(y_ref[...] * sc + sh, 0.0)
        cb = zz.shape[0]
        pooled = jnp.max(zz.reshape(cb, 8, 8, 128), axis=1)
        p2 = pooled.reshape(cb * 8, 128)
        yo = jax.lax.dot_general(p2.astype(jnp.bfloat16), wt_ref[...].astype(jnp.bfloat16),
                                 (((1,), (0,)), ((), ())),
                                 preferred_element_type=_F32)
        ncxb = ncx_ref[...].astype(jnp.bfloat16).astype(_F32)
        ncyb = ncy_ref[...].astype(jnp.bfloat16).astype(_F32)
        nczb = ncz_ref[...].astype(jnp.bfloat16).astype(_F32)
        wxb = wx_ref[...].astype(jnp.bfloat16).astype(_F32)
        wyb = wy_ref[...].astype(jnp.bfloat16).astype(_F32)
        wzb = wz_ref[...].astype(jnp.bfloat16).astype(_F32)
        yo = (yo + ncxb * wxb + ncyb * wyb + nczb * wzb + b_ref[...])
        yo_ref[...] = yo
        _accum_stats(sto_ref, _stats_block(yo))
    return body


def _make_final(n_rows):
    def body(y_ref, st_ref, g_ref, be_ref, o_ref):
        sc, sh = _scale_shift(st_ref, g_ref, be_ref, n_rows)
        zz = jnp.maximum(y_ref[...] * sc + sh, 0.0)
        r = zz.shape[0] // 8
        o_ref[...] = jnp.max(zz.reshape(r, 8, zz.shape[1]), axis=1)
    return body


def kernel(xyz, sa1_W0, sa1_b0, sa1_g0, sa1_be0, sa1_W1, sa1_b1, sa1_g1, sa1_be1,
           sa1_W2, sa1_b2, sa1_g2, sa1_be2, sa2_W0, sa2_b0, sa2_g0, sa2_be0,
           sa2_W1, sa2_b1, sa2_g1, sa2_be1, sa2_W2, sa2_b2, sa2_g2, sa2_be2):
    bs, nv, k, _ = xyz.shape
    B = bs * nv
    xt = jnp.transpose(xyz.reshape(B, k, 3), (2, 0, 1))
    px, py, pz = xt[0], xt[1], xt[2]

    cb_a = 128
    y1, ncx, ncy, ncz, st1 = pl.pallas_call(
        _group_body,
        grid=(B // cb_a,),
        in_specs=[
            pl.BlockSpec((cb_a, 32), lambda i: (i, 0)),
            pl.BlockSpec((cb_a, 32), lambda i: (i, 0)),
            pl.BlockSpec((cb_a, 32), lambda i: (i, 0)),
            pl.BlockSpec((3, 64), lambda i: (0, 0)),
            pl.BlockSpec((1, 64), lambda i: (0, 0)),
        ],
        out_specs=[
            pl.BlockSpec((cb_a, 64, 64), lambda i: (i, 0, 0)),
            pl.BlockSpec((cb_a, 8), lambda i: (i, 0)),
            pl.BlockSpec((cb_a, 8), lambda i: (i, 0)),
            pl.BlockSpec((cb_a, 8), lambda i: (i, 0)),
            pl.BlockSpec((8, 64), lambda i: (0, 0)),
        ],
        out_shape=[
            jax.ShapeDtypeStruct((B, 64, 64), _F32),
            jax.ShapeDtypeStruct((B, 8), _F32),
            jax.ShapeDtypeStruct((B, 8), _F32),
            jax.ShapeDtypeStruct((B, 8), _F32),
            jax.ShapeDtypeStruct((8, 64), _F32),
        ],
    )(px, py, pz, jnp.transpose(sa1_W0), sa1_b0.reshape(1, 64))

    def mlp(yprev, st, g, be, W, b, n_rows, rb):
        R, cin = yprev.shape
        cout = W.shape[0]
        return pl.pallas_call(
            _make_mlp(n_rows),
            grid=(R // rb,),
            in_specs=[
                pl.BlockSpec((rb, cin), lambda i: (i, 0)),
                pl.BlockSpec((8, cin), lambda i: (0, 0)),
                pl.BlockSpec((1, cin), lambda i: (0, 0)),
                pl.BlockSpec((1, cin), lambda i: (0, 0)),
                pl.BlockSpec((cin, cout), lambda i: (0, 0)),
                pl.BlockSpec((1, cout), lambda i: (0, 0)),
            ],
            out_specs=[
                pl.BlockSpec((rb, cout), lambda i: (i, 0)),
                pl.BlockSpec((8, cout), lambda i: (0, 0)),
            ],
            out_shape=[
                jax.ShapeDtypeStruct((R, cout), _F32),
                jax.ShapeDtypeStruct((8, cout), _F32),
            ],
        )(yprev, st, g.reshape(1, cin), be.reshape(1, cin),
          jnp.transpose(W), b.reshape(1, cout))

    n1 = float(B * 64)
    n2 = float(B * 8)

    y2, st2 = mlp(y1.reshape(B * 64, 64), st1, sa1_g0, sa1_be0, sa1_W1, sa1_b1, n1, 8192)
    y3, st3 = mlp(y2, st2, sa1_g1, sa1_be1, sa1_W2, sa1_b2, n1, 8192)

    cb_c = 128
    y4, st4 = pl.pallas_call(
        _make_trans(n1),
        grid=(B // cb_c,),
        in_specs=[
            pl.BlockSpec((cb_c, 64, 128), lambda i: (i, 0, 0)),
            pl.BlockSpec((8, 128), lambda i: (0, 0)),
            pl.BlockSpec((1, 128), lambda i: (0, 0)),
            pl.BlockSpec((1, 128), lambda i: (0, 0)),
            pl.BlockSpec((cb_c * 8, 1), lambda i: (i, 0)),
            pl.BlockSpec((cb_c * 8, 1), lambda i: (i, 0)),
            pl.BlockSpec((cb_c * 8, 1), lambda i: (i, 0)),
            pl.BlockSpec((1, 128), lambda i: (0, 0)),
            pl.BlockSpec((1, 128), lambda i: (0, 0)),
            pl.BlockSpec((1, 128), lambda i: (0, 0)),
            pl.BlockSpec((128, 128), lambda i: (0, 0)),
            pl.BlockSpec((1, 128), lambda i: (0, 0)),
        ],
        out_specs=[
            pl.BlockSpec((cb_c * 8, 128), lambda i: (i, 0)),
            pl.BlockSpec((8, 128), lambda i: (0, 0)),
        ],
        out_shape=[
            jax.ShapeDtypeStruct((B * 8, 128), _F32),
            jax.ShapeDtypeStruct((8, 128), _F32),
        ],
    )(y3.reshape(B, 64, 128), st3, sa1_g2.reshape(1, 128), sa1_be2.reshape(1, 128),
      ncx.reshape(B * 8, 1), ncy.reshape(B * 8, 1), ncz.reshape(B * 8, 1),
      sa2_W0[:, 0].reshape(1, 128), sa2_W0[:, 1].reshape(1, 128),
      sa2_W0[:, 2].reshape(1, 128), jnp.transpose(sa2_W0[:, 3:]),
      sa2_b0.reshape(1, 128))

    y5, st5 = mlp(y4, st4, sa2_g0, sa2_be0, sa2_W1, sa2_b1, n2, 4096)
    y6, st6 = mlp(y5, st5, sa2_g1, sa2_be1, sa2_W2, sa2_b2, n2, 2048)

    cb_d = 256
    out = pl.pallas_call(
        _make_final(n2),
        grid=(B // cb_d,),
        in_specs=[
            pl.BlockSpec((cb_d * 8, 512), lambda i: (i, 0)),
            pl.BlockSpec((8, 512), lambda i: (0, 0)),
            pl.BlockSpec((1, 512), lambda i: (0, 0)),
            pl.BlockSpec((1, 512), lambda i: (0, 0)),
        ],
        out_specs=pl.BlockSpec((cb_d, 512), lambda i: (i, 0)),
        out_shape=jax.ShapeDtypeStruct((B, 512), _F32),
    )(y6, st6, sa2_g2.reshape(1, 512), sa2_be2.reshape(1, 512))

    return out.reshape(bs, nv, 512)
